# Initial kernel scaffold; baseline (speedup 1.0000x reference)
import jax
import jax.numpy as jnp
from jax.experimental import pallas as pl


def kernel(x, last_update, edge_index, t, msg, W_time, b_time, Wq, bq, Wk, bk, Wv, bv, We, Ws, bs):
    raise NotImplementedError("write your pallas kernel here")


# speedup vs baseline: 1.0002x; 1.0002x over previous
import jax
import jax.numpy as jnp
import numpy as np
from jax.experimental import pallas as pl


def _copy_body(x_ref, o_ref):
    o_ref[...] = x_ref[...]


def kernel(x, last_update, edge_index, t, msg, W_time, b_time, Wq, bq, Wk, bk, Wv, bv, We, Ws, bs):
    src = edge_index[0]
    dst = edge_index[1]
    n = x.shape[0]
    H, C = 2, 128
    rel_t = (last_update[src] - t).astype(x.dtype)
    rel_t_enc = jnp.cos(rel_t[:, None] @ W_time + b_time)
    edge_attr = jnp.concatenate([rel_t_enc, msg], axis=-1)
    q = (x @ Wq + bq).reshape(n, H, C)[dst]
    k = (x @ Wk + bk).reshape(n, H, C)[src]
    v = (x @ Wv + bv).reshape(n, H, C)[src]
    e = (edge_attr @ We).reshape(-1, H, C)
    k = k + e
    alpha = (q * k).sum(axis=-1) / jnp.sqrt(jnp.float32(C))
    a_max = jax.ops.segment_max(alpha, dst, num_segments=n)
    a_max = jnp.where(jnp.isfinite(a_max), a_max, 0.0)
    a_exp = jnp.exp(alpha - a_max[dst])
    denom = jax.ops.segment_sum(a_exp, dst, num_segments=n)
    attn = a_exp / (denom[dst] + 1e-16)
    out_e = (v + e) * attn[..., None]
    out = jax.ops.segment_sum(out_e, dst, num_segments=n).reshape(n, H * C)
    out = out + (x @ Ws + bs)
    out = pl.pallas_call(
        _copy_body, out_shape=jax.ShapeDtypeStruct(out.shape, out.dtype)
    )(out)
    return out


# speedup vs baseline: 8.0081x; 8.0068x over previous
import functools

import jax
import jax.numpy as jnp
import numpy as np
from jax import lax
from jax.experimental import pallas as pl
from jax.experimental.pallas import tpu as pltpu
from jax.experimental.pallas import tpu_sc as plsc

N = 10000
E = 160000
D = 256
H = 2
C = 128
NC = 2
NS = 16
NW = NC * NS
INV_SQRT_C = float(1.0 / np.sqrt(C))

EPW = E // NW
EPS = E // NS
CHUNK = 80
NPAD = 10240
NROWS = NPAD // NS
RCHUNK = 128

_mesh = functools.partial(
    plsc.VectorSubcoreMesh, core_axis_name="c", subcore_axis_name="s",
    num_cores=NC, num_subcores=NS)
_mesh1 = functools.partial(
    plsc.VectorSubcoreMesh, core_axis_name="c", subcore_axis_name="s",
    num_cores=1, num_subcores=NS)
_SC_PARAMS = pltpu.CompilerParams(needs_layout_passes=False)


def _proj_body(x_ref, wq_ref, wk_ref, wv_ref, ws_ref, bq_ref, bk_ref, bv_ref,
               bs_ref, q_ref, k_ref, v_ref, s_ref):
    xb = x_ref[...]
    q_ref[...] = jnp.dot(xb, wq_ref[...], preferred_element_type=jnp.float32) + bq_ref[...]
    k_ref[...] = jnp.dot(xb, wk_ref[...], preferred_element_type=jnp.float32) + bk_ref[...]
    v_ref[...] = jnp.dot(xb, wv_ref[...], preferred_element_type=jnp.float32) + bv_ref[...]
    s_ref[...] = jnp.dot(xb, ws_ref[...], preferred_element_type=jnp.float32) + bs_ref[...]


def _projections(x, Wq, Wk, Wv, Ws, bq, bk, bv, bs):
    blk = 400
    grid = N // blk
    full = lambda shape: pl.BlockSpec(shape, lambda i: (0, 0))
    return pl.pallas_call(
        _proj_body,
        grid=(grid,),
        in_specs=[
            pl.BlockSpec((blk, D), lambda i: (i, 0)),
            full((D, H * C)), full((D, H * C)), full((D, H * C)), full((D, H * C)),
            full((1, H * C)), full((1, H * C)), full((1, H * C)), full((1, H * C)),
        ],
        out_specs=[pl.BlockSpec((blk, H * C), lambda i: (i, 0))] * 4,
        out_shape=[jax.ShapeDtypeStruct((N, H * C), jnp.float32)] * 4,
    )(x, Wq, Wk, Wv, Ws, bq.reshape(1, -1), bk.reshape(1, -1),
      bv.reshape(1, -1), bs.reshape(1, -1))


def _relt_body(lu_hbm, src_hbm, t_hbm, rel_hbm, lu_v, src_v, t_v, out_v):
    c = lax.axis_index("c")
    s = lax.axis_index("s")
    wid = c * NS + s
    base = wid * EPW
    pltpu.sync_copy(lu_hbm, lu_v)
    pltpu.sync_copy(src_hbm.at[pl.ds(base, EPW)], src_v)
    pltpu.sync_copy(t_hbm.at[pl.ds(base, EPW)], t_v)

    def vec(o):
        sv = src_v[pl.ds(o, 16)]
        lu = plsc.load_gather(lu_v, [sv])
        tv = t_v[pl.ds(o, 16)]
        out_v[pl.ds(o, 16)] = (lu - tv).astype(jnp.float32)

    def body(i, carry):
        vec(i * 16)
        return carry

    lax.fori_loop(0, EPW // 16, body, 0)
    vec(EPW - 16)
    pltpu.sync_copy(out_v, rel_hbm.at[pl.ds(base, EPW)])


def _rel_t(last_update, src, t):
    kfn = pl.kernel(
        _relt_body,
        out_type=pltpu.HBM((E,), jnp.float32),
        mesh=_mesh(),
        compiler_params=_SC_PARAMS,
        scratch_types=[
            pltpu.VMEM((N,), jnp.int32),
            pltpu.VMEM((EPW,), jnp.int32),
            pltpu.VMEM((EPW,), jnp.int32),
            pltpu.VMEM((EPW,), jnp.float32),
        ],
    )
    return kfn(last_update, src, t)


def _edge_body(rel_ref, msg_ref, wt_ref, bt_ref, we0_ref, we1_ref, e_ref):
    enc = jnp.cos(rel_ref[...] * wt_ref[...] + bt_ref[...])
    eb = (jnp.dot(enc, we0_ref[...], preferred_element_type=jnp.float32)
          + jnp.dot(msg_ref[...], we1_ref[...], preferred_element_type=jnp.float32))
    e_ref[0] = eb[:, :C]
    e_ref[1] = eb[:, C:]


def _edge_features(rel_t, msg, W_time, b_time, We):
    blk = 2000
    grid = E // blk
    T_DIM = W_time.shape[1]
    MSG_DIM = msg.shape[1]
    full = lambda shape: pl.BlockSpec(shape, lambda i: (0, 0))
    return pl.pallas_call(
        _edge_body,
        grid=(grid,),
        in_specs=[
            pl.BlockSpec((blk, 1), lambda i: (i, 0)),
            pl.BlockSpec((blk, MSG_DIM), lambda i: (i, 0)),
            full((1, T_DIM)), full((1, T_DIM)),
            full((T_DIM, H * C)), full((MSG_DIM, H * C)),
        ],
        out_specs=pl.BlockSpec((H, blk, C), lambda i: (0, i, 0)),
        out_shape=jax.ShapeDtypeStruct((H, E, C), jnp.float32),
    )(rel_t.reshape(E, 1), msg, W_time, b_time.reshape(1, -1),
      We[:T_DIM], We[T_DIM:])


def _alpha_body(q2_hbm, k2_hbm, e_hbm, src_hbm, dst_hbm,
                alpha_hbm, maxes_hbm,
                srcv, dstv, qidx, kidx, qrows, krows, erows, abuf, mbuf,
                sem0, sem1, sem2):
    c = lax.axis_index("c")
    s = lax.axis_index("s")
    wid = c * NS + s
    base = s * EPS

    def chunk(i, m):
        off = base + i * CHUNK
        pltpu.sync_copy(src_hbm.at[pl.ds(off, CHUNK)], srcv)
        pltpu.sync_copy(dst_hbm.at[pl.ds(off, CHUNK)], dstv)
        for j in range(CHUNK // 16):
            sl = pl.ds(j * 16, 16)
            qidx[sl] = dstv[sl] * 2 + c
            kidx[sl] = srcv[sl] * 2 + c
        cp0 = pltpu.async_copy(q2_hbm.at[qidx], qrows, sem0)
        cp1 = pltpu.async_copy(k2_hbm.at[kidx], krows, sem1)
        cp2 = pltpu.async_copy(e_hbm.at[c, pl.ds(off, CHUNK)], erows, sem2)
        cp0.wait()
        cp1.wait()
        cp2.wait()

        lane = lax.iota(jnp.int32, 16)

        def group(g, m):
            av = jnp.zeros((16,), jnp.float32)
            for u in range(16):
                i2 = g * 16 + u
                acc = jnp.zeros((16,), jnp.float32)
                for j in range(C // 16):
                    sl = pl.ds(j * 16, 16)
                    acc = acc + qrows[i2, sl] * (krows[i2, sl] + erows[i2, sl])
                sv = jnp.sum(acc) * INV_SQRT_C
                av = jnp.where(lane == u, sv, av)
            abuf[pl.ds(g * 16, 16)] = av
            return jnp.maximum(m, jnp.max(av))

        m = lax.fori_loop(0, CHUNK // 16, group, m)
        pltpu.sync_copy(abuf, alpha_hbm.at[pl.ds(pl.multiple_of(c * E + off, 8), CHUNK)])
        return m

    m = lax.fori_loop(0, EPS // CHUNK, chunk, jnp.float32(-3e38))
    mbuf[...] = jnp.zeros((16,), jnp.float32) + m
    pltpu.sync_copy(mbuf, maxes_hbm.at[wid])


def _alpha(q2, k2, e, src, dst):
    kfn = pl.kernel(
        _alpha_body,
        out_type=[
            pltpu.HBM((H * E,), jnp.float32),
            pltpu.HBM((NW, 16), jnp.float32),
        ],
        mesh=_mesh(),
        compiler_params=_SC_PARAMS,
        scratch_types=[
            pltpu.VMEM((CHUNK,), jnp.int32),
            pltpu.VMEM((CHUNK,), jnp.int32),
            pltpu.VMEM((CHUNK,), jnp.int32),
            pltpu.VMEM((CHUNK,), jnp.int32),
            pltpu.VMEM((CHUNK, C), jnp.float32),
            pltpu.VMEM((CHUNK, C), jnp.float32),
            pltpu.VMEM((CHUNK, C), jnp.float32),
            pltpu.VMEM((CHUNK,), jnp.float32),
            pltpu.VMEM((16,), jnp.float32),
            pltpu.SemaphoreType.DMA,
            pltpu.SemaphoreType.DMA,
            pltpu.SemaphoreType.DMA,
        ],
    )
    return kfn(q2, k2, e, src, dst)


NLOC = 5120
ACC_R = NLOC + 8
DEN_R = 48
NSUB = NLOC // NS
RCH = 32


def _scatter_body(v2_hbm, e_hbm, alpha_hbm, src_hbm, dst_hbm, m_hbm,
                  out0_hbm, out1_hbm,
                  srcv, dstv, vidx, lidx, didx, vrows, erows, abuf, wbuf,
                  contrib, dcontrib, rbuf, obuf, dball, mbuf, acc, dacc, sem0):
    c = lax.axis_index("c")
    s = lax.axis_index("s")
    base = s * EPS
    lo = c * NLOC
    pltpu.sync_copy(m_hbm, mbuf)
    m = mbuf[...][0]
    lanes = [lax.iota(jnp.int32, 16) + 16 * j for j in range(C // 16)]

    for h in range(H):
        out_hbm = out0_hbm if h == 0 else out1_hbm
        plsc.subcore_barrier()

        def zrow(r, carry):
            for j in range(C // 16):
                rbuf[r, pl.ds(j * 16, 16)] = jnp.zeros((16,), jnp.float32)
            return carry

        lax.fori_loop(0, RCH, zrow, 0)
        for k in range(NSUB // RCH):
            pltpu.sync_copy(rbuf, acc.at[pl.ds(s * NSUB + k * RCH, RCH)])

        @pl.when(s == 0)
        def _():
            pltpu.sync_copy(rbuf.at[pl.ds(0, 8)], acc.at[pl.ds(NLOC, 8)])
            pltpu.sync_copy(rbuf.at[pl.ds(0, RCH)], dacc.at[pl.ds(0, RCH)])
            pltpu.sync_copy(rbuf.at[pl.ds(0, DEN_R - RCH)], dacc.at[pl.ds(RCH, DEN_R - RCH)])

        plsc.subcore_barrier()

        def chunk(i, carry):
            off = base + i * CHUNK
            pltpu.sync_copy(src_hbm.at[pl.ds(off, CHUNK)], srcv)
            pltpu.sync_copy(dst_hbm.at[pl.ds(off, CHUNK)], dstv)
            for j in range(CHUNK // 16):
                sl = pl.ds(j * 16, 16)
                vidx[sl] = srcv[sl] * 2 + h
                dloc = dstv[sl] - lo
                keep = (dloc >= 0) & (dloc < NLOC)
                dloc = jnp.where(keep, dloc, NLOC)
                lidx[sl] = dloc
                didx[sl] = dloc >> 7
            cp0 = pltpu.async_copy(v2_hbm.at[vidx], vrows, sem0)
            pltpu.sync_copy(e_hbm.at[h, pl.ds(off, CHUNK)], erows)
            pltpu.sync_copy(alpha_hbm.at[pl.ds(h * E + off, CHUNK)], abuf)
            for j in range(CHUNK // 16):
                sl = pl.ds(j * 16, 16)
                wbuf[sl] = jnp.exp(abuf[sl] - m)
            cp0.wait()

            def group(g, carry):
                wv = wbuf[pl.ds(g * 16, 16)]
                dgv = lidx[pl.ds(g * 16, 16)] & 127
                for u in range(16):
                    i2 = g * 16 + u
                    wi = wv[u]
                    dl = dgv[u]
                    for j in range(C // 16):
                        sl = pl.ds(j * 16, 16)
                        contrib[i2, sl] = (vrows[i2, sl] + erows[i2, sl]) * wi
                        dcontrib[i2, sl] = jnp.where(lanes[j] == dl, wi, 0.0)
                return carry

            lax.fori_loop(0, CHUNK // 16, group, 0)
            pltpu.sync_copy(contrib, acc.at[lidx], add=True)
            pltpu.sync_copy(dcontrib, dacc.at[didx], add=True)
            return carry

        lax.fori_loop(0, EPS // CHUNK, chunk, 0)
        plsc.subcore_barrier()

        pltpu.sync_copy(dacc, dball)
        for k in range(NSUB // RCH):
            l0 = s * NSUB + k * RCH
            pltpu.sync_copy(acc.at[pl.ds(l0, RCH)], rbuf)

            def rgroup(g, carry):
                lb = l0 + g * 16
                dv = dball[lb >> 7, pl.ds(lb & 127, 16)]
                wvv = 1.0 / (dv + 1e-16)
                for u in range(16):
                    r = g * 16 + u
                    winv = wvv[u]
                    for j in range(C // 16):
                        sl = pl.ds(j * 16, 16)
                        obuf[r, sl] = rbuf[r, sl] * winv
                return carry

            lax.fori_loop(0, RCH // 16, rgroup, 0)
            pltpu.sync_copy(obuf, out_hbm.at[pl.ds(pl.multiple_of(lo + l0, 8), RCH)])


def _scatter(v2, e, alpha, src, dst, mvec):
    kfn = pl.kernel(
        _scatter_body,
        out_type=[
            pltpu.HBM((NPAD, C), jnp.float32),
            pltpu.HBM((NPAD, C), jnp.float32),
        ],
        mesh=_mesh(),
        compiler_params=_SC_PARAMS,
        scratch_types=[
            pltpu.VMEM((CHUNK,), jnp.int32),
            pltpu.VMEM((CHUNK,), jnp.int32),
            pltpu.VMEM((CHUNK,), jnp.int32),
            pltpu.VMEM((CHUNK,), jnp.int32),
            pltpu.VMEM((CHUNK,), jnp.int32),
            pltpu.VMEM((CHUNK, C), jnp.float32),
            pltpu.VMEM((CHUNK, C), jnp.float32),
            pltpu.VMEM((CHUNK,), jnp.float32),
            pltpu.VMEM((CHUNK,), jnp.float32),
            pltpu.VMEM((CHUNK, C), jnp.float32),
            pltpu.VMEM((CHUNK, C), jnp.float32),
            pltpu.VMEM((RCH, C), jnp.float32),
            pltpu.VMEM((RCH, C), jnp.float32),
            pltpu.VMEM((DEN_R, C), jnp.float32),
            pltpu.VMEM((16,), jnp.float32),
            pltpu.VMEM_SHARED((ACC_R, C), jnp.float32),
            pltpu.VMEM_SHARED((DEN_R, C), jnp.float32),
            pltpu.SemaphoreType.DMA,
        ],
    )
    return kfn(v2, e, alpha, src, dst, mvec)


def _combine_body(o0_ref, o1_ref, s_ref, out_ref):
    out_ref[:, :C] = o0_ref[...] + s_ref[:, :C]
    out_ref[:, C:] = o1_ref[...] + s_ref[:, C:]


def _combine(o0, o1, skip):
    blk = 400
    return pl.pallas_call(
        _combine_body,
        grid=(N // blk,),
        in_specs=[
            pl.BlockSpec((blk, C), lambda i: (i, 0)),
            pl.BlockSpec((blk, C), lambda i: (i, 0)),
            pl.BlockSpec((blk, H * C), lambda i: (i, 0)),
        ],
        out_specs=pl.BlockSpec((blk, H * C), lambda i: (i, 0)),
        out_shape=jax.ShapeDtypeStruct((N, H * C), jnp.float32),
    )(o0, o1, skip)


def kernel(x, last_update, edge_index, t, msg, W_time, b_time, Wq, bq, Wk, bk,
           Wv, bv, We, Ws, bs):
    src = edge_index[0]
    dst = edge_index[1]
    Q, K, V, S = _projections(x, Wq, Wk, Wv, Ws, bq, bk, bv, bs)
    rel_t = _rel_t(last_update, src, t)
    e = _edge_features(rel_t, msg, W_time, b_time, We)
    q2 = Q.reshape(N * H, C)
    k2 = K.reshape(N * H, C)
    v2 = V.reshape(N * H, C)
    alpha, maxes = _alpha(q2, k2, e, src, dst)
    M = jnp.max(maxes)
    mvec = jnp.full((16,), M, jnp.float32)
    o0, o1 = _scatter(v2, e, alpha, src, dst, mvec)
    return _combine(o0, o1, S)
